# baseline (speedup 1.0000x reference)
import functools

import jax
import jax.numpy as jnp
from jax import lax
from jax.experimental import pallas as pl
from jax.experimental.pallas import tpu as pltpu
from jax.experimental.pallas import tpu_sc as plsc

N_NODES = 10000
N_EDGES = 320000
D = 128
NC, NS, L = 2, 16, 16
NW = NC * NS
K = 128
DP = D // 2
CPW = 80
CR = 8
NR = CPW // CR
E_PAD = NW * CPW * K
ACC_ROWS = 10240
VR = D // L


def _sc_body(src_h, dst_h, att_h, ent_h, out_h,
             acc, srcb, dstb, attb, g0, g1,
             gs0, gs1, isem):
    c = lax.axis_index("c")
    s = lax.axis_index("s")
    w = s * NC + c
    base = w * CPW

    def stage_idx(round_no, half):
        src_slab = src_h.at[pl.ds(base + CR * round_no, CR)]
        dst_slab = dst_h.at[pl.ds(base + CR * round_no, CR)]
        att_slab = att_h.at[pl.ds(base + CR * round_no, CR)]
        pltpu.async_copy(src_slab, srcb.at[pl.ds(half, CR)], isem)
        pltpu.async_copy(dst_slab, dstb.at[pl.ds(half, CR)], isem)
        pltpu.async_copy(att_slab, attb.at[pl.ds(half, CR)], isem)

    def wait_idx(round_no, half):
        for buf in (srcb, dstb, attb):
            pltpu.make_async_copy(src_h.at[pl.ds(base + CR * round_no, CR)],
                                  buf.at[pl.ds(half, CR)], isem).wait()

    stage_idx(0, 0)
    stage_idx(1, CR)

    def zero_row(r, carry):
        for k in range(VR):
            g0[r, pl.ds(k * L, L)] = jnp.zeros((L,), jnp.float32)
        return carry
    lax.fori_loop(0, K, zero_row, 0)
    for j in range(ACC_ROWS // K // NS):
        pltpu.sync_copy(g0, acc.at[pl.ds((s * 5 + j) * K, K)])
    plsc.subcore_barrier()

    def gather(gb, gsem, slot):
        pltpu.async_copy(ent_h.at[srcb.at[slot]], gb, gsem)

    def gather_wait(gb, gsem):
        pltpu.make_async_copy(ent_h.at[srcb.at[0]], gb, gsem).wait()

    def scatter(gb, slot):
        pltpu.sync_copy(gb, acc.at[dstb.at[slot]], add=True)

    def scale(gb, slot):
        def grp(g, carry):
            av_vec = attb[slot, pl.ds(g * L, L)]
            for i in range(L):
                av = av_vec[i]
                e = g * L + i
                for k in range(VR):
                    gb[e, pl.ds(k * L, L)] = gb[e, pl.ds(k * L, L)] * av
            return carry
        lax.fori_loop(0, K // L, grp, 0)

    wait_idx(0, 0)
    gather(g0, gs0, 0)
    gather(g1, gs1, 1)

    def round_body(p, carry):
        mb = lax.rem(p, 2) * CR
        for pair in range(CR // 2):
            s0 = mb + 2 * pair
            s1 = s0 + 1
            gather_wait(g0, gs0)
            scale(g0, s0)
            scatter(g0, s0)
            if pair < CR // 2 - 1:
                gather(g0, gs0, s0 + 2)
            gather_wait(g1, gs1)
            scale(g1, s1)
            scatter(g1, s1)
            if pair < CR // 2 - 1:
                gather(g1, gs1, s1 + 2)

        @pl.when(p < NR - 1)
        def _():
            mb2 = lax.rem(p + 1, 2) * CR
            wait_idx(p + 1, mb2)
            gather(g0, gs0, mb2)
            gather(g1, gs1, mb2 + 1)

            @pl.when(p < NR - 2)
            def _():
                stage_idx(p + 2, mb)
        return carry
    lax.fori_loop(0, NR, round_body, 0)
    plsc.subcore_barrier()

    rpt = ACC_ROWS // NS
    pltpu.sync_copy(acc.at[pl.ds(s * rpt, rpt)],
                    out_h.at[c, pl.ds(s * rpt, rpt)])


_sc_call = functools.partial(
    pl.kernel,
    out_type=jax.ShapeDtypeStruct((NC, ACC_ROWS, D), jnp.float32),
    mesh=plsc.VectorSubcoreMesh(core_axis_name="c", subcore_axis_name="s",
                                num_cores=NC, num_subcores=NS),
    compiler_params=pltpu.CompilerParams(use_tc_tiling_on_sc=False),
    scratch_types=[
        pltpu.VMEM_SHARED((ACC_ROWS, D), jnp.float32),
        pltpu.VMEM((2 * CR, K), jnp.int32),
        pltpu.VMEM((2 * CR, K), jnp.int32),
        pltpu.VMEM((2 * CR, K), jnp.float32),
        pltpu.VMEM((K, D), jnp.float32),
        pltpu.VMEM((K, D), jnp.float32),
    ] + [pltpu.SemaphoreType.DMA] * 3,
)(_sc_body)


def _tc_body(ent_ref, p0_ref, p1_ref, w_ref, b_ref, out_ref):
    x = ent_ref[...] + p0_ref[...] + p1_ref[...]
    y = lax.dot_general(x, w_ref[...], (((1,), (1,)), ((), ())),
                        preferred_element_type=jnp.float32) + b_ref[...]
    out_ref[...] = jnp.where(y >= 0, y, 0.01 * y)


_TC_BLK = 2000

_tc_call = pl.pallas_call(
    _tc_body,
    grid=(N_NODES // _TC_BLK,),
    in_specs=[
        pl.BlockSpec((_TC_BLK, D), lambda i: (i, 0)),
        pl.BlockSpec((_TC_BLK, D), lambda i: (i, 0)),
        pl.BlockSpec((_TC_BLK, D), lambda i: (i, 0)),
        pl.BlockSpec((D, D), lambda i: (0, 0)),
        pl.BlockSpec((1, D), lambda i: (0, 0)),
    ],
    out_specs=pl.BlockSpec((_TC_BLK, D), lambda i: (i, 0)),
    out_shape=jax.ShapeDtypeStruct((N_NODES, D), jnp.float32),
)


def kernel(entity_embed, edge_index, edge_att, W, b):
    pad = E_PAD - N_EDGES
    src = jnp.concatenate([edge_index[0], jnp.zeros((pad,), jnp.int32)])
    dst = jnp.concatenate([edge_index[1], jnp.zeros((pad,), jnp.int32)])
    att = jnp.concatenate([edge_att.reshape(-1), jnp.zeros((pad,), jnp.float32)])
    ent_pk = entity_embed
    partial = _sc_call(src.reshape(-1, K), dst.reshape(-1, K),
                       att.reshape(-1, K), ent_pk)
    return _tc_call(entity_embed, partial[0, :N_NODES], partial[1, :N_NODES],
                    W, b.reshape(1, D))


# speedup vs baseline: 1.0035x; 1.0035x over previous
import functools

import jax
import jax.numpy as jnp
from jax import lax
from jax.experimental import pallas as pl
from jax.experimental.pallas import tpu as pltpu
from jax.experimental.pallas import tpu_sc as plsc

N_NODES = 10000
N_EDGES = 320000
D = 128
NC, NS, L = 2, 16, 16
NW = NC * NS
K = 112
DP = D // 2
CPW = 96
CR = 8
NR = CPW // CR
E_PAD = NW * CPW * K
ACC_ROWS = 10240
VR = D // L
RPT = ACC_ROWS // NS


def _sc_body(src_h, dst_h, att_h, ent_h, out_h,
             acc, srcb, dstb, attb, g0, g1, sb0, sb1,
             gs0, gs1, ss0, ss1, isem):
    gbufs = (g0, g1)
    gsems = (gs0, gs1)
    sbufs = (sb0, sb1)
    ssems = (ss0, ss1)
    c = lax.axis_index("c")
    s = lax.axis_index("s")
    w = s * NC + c
    base = w * CPW

    def stage_idx(round_no, half):
        src_slab = src_h.at[pl.ds(base + CR * round_no, CR)]
        dst_slab = dst_h.at[pl.ds(base + CR * round_no, CR)]
        att_slab = att_h.at[pl.ds(base + CR * round_no, CR)]
        pltpu.async_copy(src_slab, srcb.at[pl.ds(half, CR)], isem)
        pltpu.async_copy(dst_slab, dstb.at[pl.ds(half, CR)], isem)
        pltpu.async_copy(att_slab, attb.at[pl.ds(half, CR)], isem)

    def wait_idx(round_no, half):
        for buf in (srcb, dstb, attb):
            pltpu.make_async_copy(src_h.at[pl.ds(base + CR * round_no, CR)],
                                  buf.at[pl.ds(half, CR)], isem).wait()

    stage_idx(0, 0)
    stage_idx(1, CR)

    def zero_row(r, carry):
        for k in range(VR):
            sb0[r, pl.ds(k * L, L)] = jnp.zeros((L,), jnp.float32)
        return carry
    lax.fori_loop(0, K, zero_row, 0)
    for j in range(RPT // K):
        pltpu.sync_copy(sb0, acc.at[pl.ds(s * RPT + j * K, K)])
    pltpu.sync_copy(sb0.at[pl.ds(0, RPT - (RPT // K) * K)],
                    acc.at[pl.ds(s * RPT + (RPT // K) * K,
                                 RPT - (RPT // K) * K)])
    plsc.subcore_barrier()

    def gather(b, slot):
        pltpu.async_copy(ent_h.at[srcb.at[slot]], gbufs[b], gsems[b])

    def gather_wait(b):
        pltpu.make_async_copy(ent_h.at[srcb.at[0]], gbufs[b], gsems[b]).wait()

    def scatter(b, slot):
        pltpu.async_copy(sbufs[b], acc.at[dstb.at[slot]], ssems[b], add=True)

    def scatter_wait(b):
        pltpu.make_async_copy(sbufs[b], acc.at[dstb.at[0]], ssems[b]).wait()

    def scale(b, slot):
        gb = gbufs[b]
        sb = sbufs[b]

        def grp(g, carry):
            av_vec = attb[slot, pl.ds(g * L, L)]
            for i in range(L):
                av = av_vec[i]
                e = g * L + i
                for k in range(VR // 2):
                    pk = gb[e, pl.ds(k * L, L)]
                    lo = lax.bitcast_convert_type(
                        lax.shift_left(pk, 16), jnp.float32)
                    hi = lax.bitcast_convert_type(
                        lax.bitwise_and(pk, jnp.int32(-65536)), jnp.float32)
                    sb[e, pl.ds(k * 2 * L, L)] = lo * av
                    sb[e, pl.ds(k * 2 * L + L, L)] = hi * av
            return carry
        lax.fori_loop(0, K // L, grp, 0)

    wait_idx(0, 0)
    gather(0, 0)
    gather(1, 1)

    def round_body(p, carry):
        mb = lax.rem(p, 2) * CR
        for q in range(CR):
            b = q % 2
            slot = mb + q
            gather_wait(b)
            if q >= 2:
                scatter_wait(b)
            else:
                @pl.when(p > 0)
                def _():
                    scatter_wait(b)
            scale(b, slot)
            scatter(b, slot)
            if q < CR - 2:
                gather(b, slot + 2)

        @pl.when(p < NR - 1)
        def _():
            mb2 = lax.rem(p + 1, 2) * CR
            wait_idx(p + 1, mb2)
            gather(0, mb2)
            gather(1, mb2 + 1)

            @pl.when(p < NR - 2)
            def _():
                stage_idx(p + 2, mb)
        return carry
    lax.fori_loop(0, NR, round_body, 0)
    scatter_wait(0)
    scatter_wait(1)
    plsc.subcore_barrier()

    pltpu.sync_copy(acc.at[pl.ds(s * RPT, RPT)],
                    out_h.at[c, pl.ds(s * RPT, RPT)])


_sc_call = functools.partial(
    pl.kernel,
    out_type=jax.ShapeDtypeStruct((NC, ACC_ROWS, D), jnp.float32),
    mesh=plsc.VectorSubcoreMesh(core_axis_name="c", subcore_axis_name="s",
                                num_cores=NC, num_subcores=NS),
    compiler_params=pltpu.CompilerParams(use_tc_tiling_on_sc=False),
    scratch_types=[
        pltpu.VMEM_SHARED((ACC_ROWS, D), jnp.float32),
        pltpu.VMEM((2 * CR, K), jnp.int32),
        pltpu.VMEM((2 * CR, K), jnp.int32),
        pltpu.VMEM((2 * CR, K), jnp.float32),
        pltpu.VMEM((K, DP), jnp.int32),
        pltpu.VMEM((K, DP), jnp.int32),
        pltpu.VMEM((K, D), jnp.float32),
        pltpu.VMEM((K, D), jnp.float32),
    ] + [pltpu.SemaphoreType.DMA] * 5,
)(_sc_body)


def _tc_body(ent_ref, p0_ref, p1_ref, w_ref, b_ref, out_ref):
    x = ent_ref[...] + p0_ref[...] + p1_ref[...]
    y = lax.dot_general(x, w_ref[...], (((1,), (1,)), ((), ())),
                        preferred_element_type=jnp.float32) + b_ref[...]
    out_ref[...] = jnp.where(y >= 0, y, 0.01 * y)


_TC_BLK = 2000

_tc_call = pl.pallas_call(
    _tc_body,
    grid=(N_NODES // _TC_BLK,),
    in_specs=[
        pl.BlockSpec((_TC_BLK, D), lambda i: (i, 0)),
        pl.BlockSpec((_TC_BLK, D), lambda i: (i, 0)),
        pl.BlockSpec((_TC_BLK, D), lambda i: (i, 0)),
        pl.BlockSpec((D, D), lambda i: (0, 0)),
        pl.BlockSpec((1, D), lambda i: (0, 0)),
    ],
    out_specs=pl.BlockSpec((_TC_BLK, D), lambda i: (i, 0)),
    out_shape=jax.ShapeDtypeStruct((N_NODES, D), jnp.float32),
)


def kernel(entity_embed, edge_index, edge_att, W, b):
    pad = E_PAD - N_EDGES
    src = jnp.concatenate([edge_index[0], jnp.zeros((pad,), jnp.int32)])
    dst = jnp.concatenate([edge_index[1], jnp.zeros((pad,), jnp.int32)])
    att = jnp.concatenate([edge_att.reshape(-1), jnp.zeros((pad,), jnp.float32)])
    e16 = entity_embed.astype(jnp.bfloat16)
    m = e16.reshape(N_NODES, 4, 2, L).transpose(0, 1, 3, 2).reshape(N_NODES, DP, 2)
    ent_pk = jax.lax.bitcast_convert_type(m, jnp.int32)
    partial = _sc_call(src.reshape(-1, K), dst.reshape(-1, K),
                       att.reshape(-1, K), ent_pk)
    return _tc_call(entity_embed, partial[0, :N_NODES], partial[1, :N_NODES],
                    W, b.reshape(1, D))


# speedup vs baseline: 1.4150x; 1.4100x over previous
import functools

import jax
import jax.numpy as jnp
from jax import lax
from jax.experimental import pallas as pl
from jax.experimental.pallas import tpu as pltpu
from jax.experimental.pallas import tpu_sc as plsc

N_NODES = 10000
N_EDGES = 320000
D = 128
NC, NS, L = 2, 16, 16
NW = NC * NS
K = 128
DP = D // 2
CPW = 80
CR = 8
NR = CPW // CR
E_PAD = NW * CPW * K
ACC_ROWS = 10240
VR = D // L


def _sc_body(src_h, dst_h, att_h, ent_h, out_h,
             acc, srcb, dstb, attb, g0, g1, sbuf,
             gs0, gs1, isem):
    c = lax.axis_index("c")
    s = lax.axis_index("s")
    w = s * NC + c
    base = w * CPW

    def stage_idx(round_no, half):
        src_slab = src_h.at[pl.ds(base + CR * round_no, CR)]
        dst_slab = dst_h.at[pl.ds(base + CR * round_no, CR)]
        att_slab = att_h.at[pl.ds(base + CR * round_no, CR)]
        pltpu.async_copy(src_slab, srcb.at[pl.ds(half, CR)], isem)
        pltpu.async_copy(dst_slab, dstb.at[pl.ds(half, CR)], isem)
        pltpu.async_copy(att_slab, attb.at[pl.ds(half, CR)], isem)

    def wait_idx(round_no, half):
        for buf in (srcb, dstb, attb):
            pltpu.make_async_copy(src_h.at[pl.ds(base + CR * round_no, CR)],
                                  buf.at[pl.ds(half, CR)], isem).wait()

    stage_idx(0, 0)
    stage_idx(1, CR)

    def zero_row(r, carry):
        for k in range(VR):
            sbuf[r, pl.ds(k * L, L)] = jnp.zeros((L,), jnp.float32)
        return carry
    lax.fori_loop(0, K, zero_row, 0)
    for j in range(ACC_ROWS // K // NS):
        pltpu.sync_copy(sbuf, acc.at[pl.ds((s * 5 + j) * K, K)])
    plsc.subcore_barrier()

    def gather(gb, gsem, slot):
        pltpu.async_copy(ent_h.at[srcb.at[slot]], gb, gsem)

    def gather_wait(gb, gsem):
        pltpu.make_async_copy(ent_h.at[srcb.at[0]], gb, gsem).wait()

    def scatter(slot):
        pltpu.sync_copy(sbuf, acc.at[dstb.at[slot]], add=True)

    def scale(gb, slot):
        def grp(g, carry):
            av_vec = attb[slot, pl.ds(g * L, L)]
            for i in range(L):
                av = av_vec[i]
                e = g * L + i
                for k in range(VR // 2):
                    pk = gb[e, pl.ds(k * L, L)]
                    lo = lax.bitcast_convert_type(
                        lax.shift_left(pk, 16), jnp.float32)
                    hi = lax.bitcast_convert_type(
                        lax.bitwise_and(pk, jnp.int32(-65536)), jnp.float32)
                    sbuf[e, pl.ds(k * 2 * L, L)] = lo * av
                    sbuf[e, pl.ds(k * 2 * L + L, L)] = hi * av
            return carry
        lax.fori_loop(0, K // L, grp, 0)

    wait_idx(0, 0)
    gather(g0, gs0, 0)
    gather(g1, gs1, 1)

    def round_body(p, carry):
        mb = lax.rem(p, 2) * CR
        for pair in range(CR // 2):
            s0 = mb + 2 * pair
            s1 = s0 + 1
            gather_wait(g0, gs0)
            scale(g0, s0)
            if pair < CR // 2 - 1:
                gather(g0, gs0, s0 + 2)
            scatter(s0)
            gather_wait(g1, gs1)
            scale(g1, s1)
            if pair < CR // 2 - 1:
                gather(g1, gs1, s1 + 2)
            scatter(s1)

        @pl.when(p < NR - 1)
        def _():
            mb2 = lax.rem(p + 1, 2) * CR
            wait_idx(p + 1, mb2)
            gather(g0, gs0, mb2)
            gather(g1, gs1, mb2 + 1)

            @pl.when(p < NR - 2)
            def _():
                stage_idx(p + 2, mb)
        return carry
    lax.fori_loop(0, NR, round_body, 0)
    plsc.subcore_barrier()

    rpt = ACC_ROWS // NS
    pltpu.sync_copy(acc.at[pl.ds(s * rpt, rpt)],
                    out_h.at[c, pl.ds(s * rpt, rpt)])


_sc_call = functools.partial(
    pl.kernel,
    out_type=jax.ShapeDtypeStruct((NC, ACC_ROWS, D), jnp.float32),
    mesh=plsc.VectorSubcoreMesh(core_axis_name="c", subcore_axis_name="s",
                                num_cores=NC, num_subcores=NS),
    compiler_params=pltpu.CompilerParams(use_tc_tiling_on_sc=False),
    scratch_types=[
        pltpu.VMEM_SHARED((ACC_ROWS, D), jnp.float32),
        pltpu.VMEM((2 * CR, K), jnp.int32),
        pltpu.VMEM((2 * CR, K), jnp.int32),
        pltpu.VMEM((2 * CR, K), jnp.float32),
        pltpu.VMEM((K, DP), jnp.int32),
        pltpu.VMEM((K, DP), jnp.int32),
        pltpu.VMEM((K, D), jnp.float32),
    ] + [pltpu.SemaphoreType.DMA] * 3,
)(_sc_body)


def _tc_body(ent_ref, p0_ref, p1_ref, w_ref, b_ref, out_ref):
    x = ent_ref[...] + p0_ref[...] + p1_ref[...]
    y = lax.dot_general(x, w_ref[...], (((1,), (1,)), ((), ())),
                        preferred_element_type=jnp.float32) + b_ref[...]
    out_ref[...] = jnp.where(y >= 0, y, 0.01 * y)


_TC_BLK = 2000

_tc_call = pl.pallas_call(
    _tc_body,
    grid=(N_NODES // _TC_BLK,),
    in_specs=[
        pl.BlockSpec((_TC_BLK, D), lambda i: (i, 0)),
        pl.BlockSpec((_TC_BLK, D), lambda i: (i, 0)),
        pl.BlockSpec((_TC_BLK, D), lambda i: (i, 0)),
        pl.BlockSpec((D, D), lambda i: (0, 0)),
        pl.BlockSpec((1, D), lambda i: (0, 0)),
    ],
    out_specs=pl.BlockSpec((_TC_BLK, D), lambda i: (i, 0)),
    out_shape=jax.ShapeDtypeStruct((N_NODES, D), jnp.float32),
)


def kernel(entity_embed, edge_index, edge_att, W, b):
    pad = E_PAD - N_EDGES
    src = jnp.concatenate([edge_index[0], jnp.zeros((pad,), jnp.int32)])
    dst = jnp.concatenate([edge_index[1], jnp.zeros((pad,), jnp.int32)])
    att = jnp.concatenate([edge_att.reshape(-1), jnp.zeros((pad,), jnp.float32)])
    e16 = entity_embed.astype(jnp.bfloat16)
    m = e16.reshape(N_NODES, 4, 2, L).transpose(0, 1, 3, 2).reshape(N_NODES, DP, 2)
    ent_pk = jax.lax.bitcast_convert_type(m, jnp.int32)
    partial = _sc_call(src.reshape(-1, K), dst.reshape(-1, K),
                       att.reshape(-1, K), ent_pk)
    return _tc_call(entity_embed, partial[0, :N_NODES], partial[1, :N_NODES],
                    W, b.reshape(1, D))
